# baseline (speedup 1.0000x reference)
import functools

import jax
import jax.numpy as jnp
from jax import lax
from jax.experimental import pallas as pl
from jax.experimental.pallas import tpu as pltpu
from jax.experimental.pallas import tpu_sc as plsc

NUM_EMB = 8192
DIM = 256
N_TOK = 4096
COMMIT = 0.25

BN = 256
NB = N_TOK // BN
L = 16
OH_ROWS = N_TOK * (NUM_EMB // L)


def _fused_body(esq_ref, x_ref, e_ref,
                codes_ref, loss_ref, oh_ref,
                acc_ref, colf_ref):
    n = pl.program_id(0)

    @pl.when(n == 0)
    def _init_iota():
        col = lax.broadcasted_iota(jnp.int32, (BN, NUM_EMB), 1)
        colf_ref[...] = col.astype(jnp.float32)

    xv = x_ref[...]
    xsq = jnp.sum(xv * xv, axis=1, keepdims=True)
    x2 = xv * 2.0
    mm2 = lax.dot_general(x2, e_ref[...],
                          (((1,), (1,)), ((), ())),
                          preferred_element_type=jnp.float32)
    d = (xsq + esq_ref[...]) - mm2
    dmin = jnp.min(d, axis=1, keepdims=True)
    colf = colf_ref[...]
    aif = jnp.min(jnp.where(d == dmin, colf, float(NUM_EMB)), axis=1,
                  keepdims=True)
    ai = aif.astype(jnp.int32)
    codes_ref[...] = ai.reshape(1, 1, BN)
    blk_loss = jnp.sum(dmin, axis=0, keepdims=True)
    prev = acc_ref[...]
    new_acc = jnp.where(n == 0, jnp.zeros_like(prev), prev) + blk_loss
    acc_ref[...] = new_acc
    loss_ref[...] = new_acc * (COMMIT / (N_TOK * DIM))

    oh_ref[...] = jnp.zeros((BN, NUM_EMB), jnp.float32)


def _sc_body(bpw, n_cores, n_sub,
             emb_hbm, idx_hbm, oh_hbm, q_hbm, cnt_hbm,
             idx_v, rows_v, ones_v, rowidx_v, zeros_v, cnt_sp,
             sem_g, sem_s):
    cid = lax.axis_index("c")
    sid = lax.axis_index("s")
    wid = sid * n_cores + cid
    base = wid * bpw
    n_grp = bpw // L

    pltpu.sync_copy(idx_hbm.at[pl.ds(base, bpw)], idx_v)
    gather = pltpu.async_copy(emb_hbm.at[idx_v], rows_v, sem_g)

    lane = lax.broadcasted_iota(jnp.int32, (L,), 0)
    ones = jnp.ones((L,), jnp.float32)
    zero16 = jnp.zeros((L,), jnp.float32)
    for g in range(n_grp):
        c = idx_v[pl.ds(g * L, L)]
        rowidx_v[pl.ds(g * L, L)] = (base + g * L + lane) * NUM_EMB + c
        ones_v[pl.ds(g * L, L)] = ones
    scat = pltpu.async_copy(ones_v, oh_hbm.at[rowidx_v], sem_s)

    for j in range(NUM_EMB // n_sub // L):
        zeros_v[pl.ds(j * L, L)] = zero16
    pltpu.sync_copy(zeros_v, cnt_sp.at[pl.ds(sid * (NUM_EMB // n_sub),
                                             NUM_EMB // n_sub)])
    plsc.subcore_barrier()
    pltpu.sync_copy(ones_v, cnt_sp.at[idx_v], add=True)
    plsc.subcore_barrier()

    @pl.when(sid == 0)
    def _cnt_out():
        pltpu.sync_copy(cnt_sp, cnt_hbm.at[cid])

    scat.wait()
    gather.wait()
    pltpu.sync_copy(rows_v, q_hbm.at[pl.ds(base, bpw)])


def _perp_body(cnt_ref, perp_ref):
    tot = cnt_ref[0:1, :] + cnt_ref[1:2, :]
    p = tot * (1.0 / N_TOK)
    ent = jnp.sum(p * jnp.log(p + 1e-10), axis=1, keepdims=True)
    perp_ref[...] = jnp.exp(-ent)


def kernel(x, embedding):
    x_flat = x.reshape(-1, DIM)
    e_sq = jnp.sum(embedding ** 2, axis=1)

    codes3, loss2, ohz = pl.pallas_call(
        _fused_body,
        grid=(NB,),
        in_specs=[
            pl.BlockSpec((1, NUM_EMB), lambda n: (0, 0)),
            pl.BlockSpec((BN, DIM), lambda n: (n, 0)),
            pl.BlockSpec((NUM_EMB, DIM), lambda n: (0, 0)),
        ],
        out_specs=[
            pl.BlockSpec((1, 1, BN), lambda n: (n, 0, 0)),
            pl.BlockSpec((1, 1), lambda n: (0, 0)),
            pl.BlockSpec((BN, NUM_EMB), lambda n: (n, 0)),
        ],
        out_shape=[
            jax.ShapeDtypeStruct((NB, 1, BN), jnp.int32),
            jax.ShapeDtypeStruct((1, 1), jnp.float32),
            jax.ShapeDtypeStruct((N_TOK, NUM_EMB), jnp.float32),
        ],
        scratch_shapes=[
            pltpu.VMEM((1, 1), jnp.float32),
            pltpu.VMEM((BN, NUM_EMB), jnp.float32),
        ],
        compiler_params=pltpu.CompilerParams(
            dimension_semantics=("arbitrary",)),
    )(e_sq.reshape(1, NUM_EMB), x_flat, embedding)

    codes_flat = codes3.reshape(N_TOK)

    info = plsc.get_sparse_core_info()
    nc, ns = info.num_cores, info.num_subcores
    nw = nc * ns
    bpw = N_TOK // nw

    oh_ref = jax.new_ref(ohz.reshape(N_TOK * NUM_EMB))
    quantized, counts = pl.kernel(
        functools.partial(_sc_body, bpw, nc, ns),
        mesh=plsc.VectorSubcoreMesh(core_axis_name="c", subcore_axis_name="s"),
        out_type=[
            jax.ShapeDtypeStruct((N_TOK, DIM), jnp.float32),
            jax.ShapeDtypeStruct((nc, NUM_EMB), jnp.float32),
        ],
        scratch_types=[
            pltpu.VMEM((bpw,), jnp.int32),
            pltpu.VMEM((bpw, DIM), jnp.float32),
            pltpu.VMEM((bpw,), jnp.float32),
            pltpu.VMEM((bpw,), jnp.int32),
            pltpu.VMEM((NUM_EMB // ns,), jnp.float32),
            pltpu.VMEM_SHARED((NUM_EMB,), jnp.float32),
            pltpu.SemaphoreType.DMA,
            pltpu.SemaphoreType.DMA,
        ],
    )(embedding, codes_flat, oh_ref)

    perp2 = pl.pallas_call(
        _perp_body,
        grid=(1,),
        in_specs=[pl.BlockSpec((2, NUM_EMB), lambda i: (0, 0))],
        out_specs=pl.BlockSpec((1, 1), lambda i: (0, 0)),
        out_shape=jax.ShapeDtypeStruct((1, 1), jnp.float32),
    )(counts)

    B, T, _ = x.shape
    codes = codes3.reshape(B, T)
    quantized_st = quantized.reshape(x.shape)
    one_hot = oh_ref[...].reshape(B, T, NUM_EMB)
    loss = loss2[0, 0]
    perplexity = perp2[0, 0]
    return quantized_st, codes, one_hot, loss, perplexity


# speedup vs baseline: 3.3655x; 3.3655x over previous
import functools

import jax
import jax.numpy as jnp
from jax import lax
from jax.experimental import pallas as pl
from jax.experimental.pallas import tpu as pltpu
from jax.experimental.pallas import tpu_sc as plsc

NUM_EMB = 8192
DIM = 256
N_TOK = 4096
COMMIT = 0.25

BN = 256
NB = N_TOK // BN


def _fused_body(esq_ref, x_ref, e_ref,
                codes_ref, loss_ref, oh_ref, perp_ref,
                cnt_ref, acc_ref, colf_ref):
    n = pl.program_id(0)

    @pl.when(n == 0)
    def _init_iota():
        col = lax.broadcasted_iota(jnp.int32, (BN, NUM_EMB), 1)
        colf_ref[...] = col.astype(jnp.float32)

    xv = x_ref[...]
    xsq = jnp.sum(xv * xv, axis=1, keepdims=True)
    x2 = xv * 2.0
    mm2 = lax.dot_general(x2, e_ref[...],
                          (((1,), (1,)), ((), ())),
                          preferred_element_type=jnp.float32)
    esq = esq_ref[...]
    dmin = jnp.min((xsq + esq) - mm2, axis=1, keepdims=True)
    colf = colf_ref[...]
    aif = jnp.min(jnp.where(((xsq + esq) - mm2) == dmin, colf,
                            float(NUM_EMB)), axis=1,
                  keepdims=True)
    ai = aif.astype(jnp.int32)
    codes_ref[...] = ai.reshape(1, 1, BN)
    blk_loss = jnp.sum(dmin, axis=0, keepdims=True)
    prev = acc_ref[...]
    new_acc = jnp.where(n == 0, jnp.zeros_like(prev), prev) + blk_loss
    acc_ref[...] = new_acc
    loss_ref[...] = new_acc * (COMMIT / (N_TOK * DIM))

    oh = (colf == aif).astype(jnp.float32)
    oh_ref[...] = oh
    colsum = lax.dot_general(jnp.ones((1, BN), jnp.float32), oh,
                             (((1,), (0,)), ((), ())),
                             preferred_element_type=jnp.float32)
    prev_c = cnt_ref[...]
    new_cnt = jnp.where(n == 0, jnp.zeros_like(prev_c), prev_c) + colsum
    cnt_ref[...] = new_cnt

    @pl.when(n == NB - 1)
    def _fin_all():
        p = new_cnt * (1.0 / N_TOK)
        ent = jnp.sum(p * jnp.log(p + 1e-10), axis=1, keepdims=True)
        perp_ref[...] = jnp.exp(-ent)


def _sc_gather_body(bpw, n_cores, emb_hbm, idx_hbm, out_hbm,
                    idx_v, rows_v, sem):
    wid = lax.axis_index("s") * n_cores + lax.axis_index("c")
    base = wid * bpw
    pltpu.sync_copy(idx_hbm.at[pl.ds(base, bpw)], idx_v)
    pltpu.async_copy(emb_hbm.at[idx_v], rows_v, sem).wait()
    pltpu.sync_copy(rows_v, out_hbm.at[pl.ds(base, bpw)])


def kernel(x, embedding):
    x_flat = x.reshape(-1, DIM)
    e_sq = jnp.sum(embedding ** 2, axis=1)

    codes3, loss2, one_hot2, perp2 = pl.pallas_call(
        _fused_body,
        grid=(NB,),
        in_specs=[
            pl.BlockSpec((1, NUM_EMB), lambda n: (0, 0)),
            pl.BlockSpec((BN, DIM), lambda n: (n, 0)),
            pl.BlockSpec((NUM_EMB, DIM), lambda n: (0, 0)),
        ],
        out_specs=[
            pl.BlockSpec((1, 1, BN), lambda n: (n, 0, 0)),
            pl.BlockSpec((1, 1), lambda n: (0, 0)),
            pl.BlockSpec((BN, NUM_EMB), lambda n: (n, 0)),
            pl.BlockSpec((1, 1), lambda n: (0, 0)),
        ],
        out_shape=[
            jax.ShapeDtypeStruct((NB, 1, BN), jnp.int32),
            jax.ShapeDtypeStruct((1, 1), jnp.float32),
            jax.ShapeDtypeStruct((N_TOK, NUM_EMB), jnp.float32),
            jax.ShapeDtypeStruct((1, 1), jnp.float32),
        ],
        scratch_shapes=[
            pltpu.VMEM((1, NUM_EMB), jnp.float32),
            pltpu.VMEM((1, 1), jnp.float32),
            pltpu.VMEM((BN, NUM_EMB), jnp.float32),
        ],
        compiler_params=pltpu.CompilerParams(
            dimension_semantics=("arbitrary",)),
    )(e_sq.reshape(1, NUM_EMB), x_flat, embedding)

    codes_flat = codes3.reshape(N_TOK)

    info = plsc.get_sparse_core_info()
    nw = info.num_cores * info.num_subcores
    bpw = N_TOK // nw
    quantized = pl.kernel(
        functools.partial(_sc_gather_body, bpw, info.num_cores),
        mesh=plsc.VectorSubcoreMesh(core_axis_name="c", subcore_axis_name="s"),
        out_type=jax.ShapeDtypeStruct((N_TOK, DIM), jnp.float32),
        scratch_types=[
            pltpu.VMEM((bpw,), jnp.int32),
            pltpu.VMEM((bpw, DIM), jnp.float32),
            pltpu.SemaphoreType.DMA,
        ],
    )(embedding, codes_flat)

    B, T, _ = x.shape
    codes = codes3.reshape(B, T)
    quantized_st = quantized.reshape(x.shape)
    one_hot = one_hot2.reshape(B, T, NUM_EMB)
    loss = loss2[0, 0]
    perplexity = perp2[0, 0]
    return quantized_st, codes, one_hot, loss, perplexity


# speedup vs baseline: 3.4302x; 1.0192x over previous
import functools

import jax
import jax.numpy as jnp
from jax import lax
from jax.experimental import pallas as pl
from jax.experimental.pallas import tpu as pltpu
from jax.experimental.pallas import tpu_sc as plsc

NUM_EMB = 8192
DIM = 256
N_TOK = 4096
COMMIT = 0.25

BN = 256
NB = N_TOK // BN


def _fused_body(esq_ref, x_ref, e_ref,
                codes_ref, loss_ref, oh_ref, perp_ref,
                cnt_ref, acc_ref, colf_ref):
    n = pl.program_id(0)

    @pl.when(n == 0)
    def _init_iota():
        col = lax.broadcasted_iota(jnp.int32, (BN, NUM_EMB), 1)
        colf_ref[...] = col.astype(jnp.float32)

    xv = x_ref[...]
    xsq = jnp.sum(xv * xv, axis=1, keepdims=True)
    x2 = xv * 2.0
    mm2 = lax.dot_general(x2, e_ref[...],
                          (((1,), (1,)), ((), ())),
                          preferred_element_type=jnp.float32)
    esq = esq_ref[...]
    dmin = jnp.min((xsq + esq) - mm2, axis=1, keepdims=True)
    colf = colf_ref[...]
    aif = jnp.min(jnp.where(((xsq + esq) - mm2) == dmin, colf,
                            float(NUM_EMB)), axis=1,
                  keepdims=True)
    ai = aif.astype(jnp.int32)
    codes_ref[...] = ai.reshape(1, 1, BN)
    blk_loss = jnp.sum(dmin, axis=0, keepdims=True)
    prev = acc_ref[...]
    new_acc = jnp.where(n == 0, jnp.zeros_like(prev), prev) + blk_loss
    acc_ref[...] = new_acc
    loss_ref[...] = new_acc * (COMMIT / (N_TOK * DIM))

    col_i = lax.broadcasted_iota(jnp.int32, (BN, NUM_EMB), 1)
    oh = (col_i == ai).astype(jnp.float32)
    oh_ref[...] = oh
    colsum = lax.dot_general(jnp.ones((1, BN), jnp.float32), oh,
                             (((1,), (0,)), ((), ())),
                             preferred_element_type=jnp.float32)
    prev_c = cnt_ref[...]
    new_cnt = jnp.where(n == 0, jnp.zeros_like(prev_c), prev_c) + colsum
    cnt_ref[...] = new_cnt

    @pl.when(n == NB - 1)
    def _fin_all():
        p = new_cnt * (1.0 / N_TOK)
        ent = jnp.sum(p * jnp.log(p + 1e-10), axis=1, keepdims=True)
        perp_ref[...] = jnp.exp(-ent)


def _sc_gather_body(bpw, n_cores, emb_hbm, idx_hbm, out_hbm,
                    idx_v, rows_v, sem):
    wid = lax.axis_index("s") * n_cores + lax.axis_index("c")
    base = wid * bpw
    pltpu.sync_copy(idx_hbm.at[pl.ds(base, bpw)], idx_v)
    pltpu.async_copy(emb_hbm.at[idx_v], rows_v, sem).wait()
    pltpu.sync_copy(rows_v, out_hbm.at[pl.ds(base, bpw)])


def kernel(x, embedding):
    x_flat = x.reshape(-1, DIM)
    e_sq = jnp.sum(embedding ** 2, axis=1)

    codes3, loss2, one_hot2, perp2 = pl.pallas_call(
        _fused_body,
        grid=(NB,),
        in_specs=[
            pl.BlockSpec((1, NUM_EMB), lambda n: (0, 0)),
            pl.BlockSpec((BN, DIM), lambda n: (n, 0)),
            pl.BlockSpec((NUM_EMB, DIM), lambda n: (0, 0)),
        ],
        out_specs=[
            pl.BlockSpec((1, 1, BN), lambda n: (n, 0, 0)),
            pl.BlockSpec((1, 1), lambda n: (0, 0)),
            pl.BlockSpec((BN, NUM_EMB), lambda n: (n, 0)),
            pl.BlockSpec((1, 1), lambda n: (0, 0)),
        ],
        out_shape=[
            jax.ShapeDtypeStruct((NB, 1, BN), jnp.int32),
            jax.ShapeDtypeStruct((1, 1), jnp.float32),
            jax.ShapeDtypeStruct((N_TOK, NUM_EMB), jnp.float32),
            jax.ShapeDtypeStruct((1, 1), jnp.float32),
        ],
        scratch_shapes=[
            pltpu.VMEM((1, NUM_EMB), jnp.float32),
            pltpu.VMEM((1, 1), jnp.float32),
            pltpu.VMEM((BN, NUM_EMB), jnp.float32),
        ],
        compiler_params=pltpu.CompilerParams(
            dimension_semantics=("arbitrary",)),
    )(e_sq.reshape(1, NUM_EMB), x_flat, embedding)

    codes_flat = codes3.reshape(N_TOK)

    info = plsc.get_sparse_core_info()
    nw = info.num_cores * info.num_subcores
    bpw = N_TOK // nw
    quantized = pl.kernel(
        functools.partial(_sc_gather_body, bpw, info.num_cores),
        mesh=plsc.VectorSubcoreMesh(core_axis_name="c", subcore_axis_name="s"),
        out_type=jax.ShapeDtypeStruct((N_TOK, DIM), jnp.float32),
        scratch_types=[
            pltpu.VMEM((bpw,), jnp.int32),
            pltpu.VMEM((bpw, DIM), jnp.float32),
            pltpu.SemaphoreType.DMA,
        ],
    )(embedding, codes_flat)

    B, T, _ = x.shape
    codes = codes3.reshape(B, T)
    quantized_st = quantized.reshape(x.shape)
    one_hot = one_hot2.reshape(B, T, NUM_EMB)
    loss = loss2[0, 0]
    perplexity = perp2[0, 0]
    return quantized_st, codes, one_hot, loss, perplexity
